# baseline (speedup 1.0000x reference)
import functools

import jax
import jax.numpy as jnp
from jax import lax
from jax.experimental import pallas as pl
from jax.experimental.pallas import tpu as pltpu
from jax.experimental.pallas import tpu_sc as plsc

ROWS, COLS = 1024, 32768
NC, NS, L = 2, 16, 16
NW = NC * NS
ROWS_PER_W = ROWS // NW
G = 4
NGRP = ROWS_PER_W // G
CHUNK = 4096
NCH = COLS // CHUNK
BLK = 16 * L
NB = CHUNK // BLK

_MESH = plsc.VectorSubcoreMesh(
    core_axis_name="c", subcore_axis_name="s", num_cores=NC, num_subcores=NS
)


@functools.partial(
    pl.kernel,
    out_type=jax.ShapeDtypeStruct((ROWS, COLS), jnp.float32),
    mesh=_MESH,
    scratch_types=[
        pltpu.VMEM((2, G, CHUNK), jnp.float32),
        pltpu.VMEM((2, G, CHUNK // 4), jnp.int32),
        pltpu.VMEM((2, G, CHUNK), jnp.float32),
        pltpu.SemaphoreType.DMA,
        pltpu.SemaphoreType.DMA,
        pltpu.SemaphoreType.DMA,
    ],
    compiler_params=pltpu.CompilerParams(needs_layout_passes=False),
)
def _masked_cumsum_sc(x_hbm, m_hbm, out_hbm, xbuf, mbuf, obuf,
                      sem0, sem1, sem_out):
    wid = lax.axis_index("s") * NC + lax.axis_index("c")
    base_row = wid * ROWS_PER_W
    last = jnp.full((L,), L - 1, jnp.int32)
    lane_c = [jnp.full((L,), j, jnp.int32) for j in range(L)]
    iota16 = lax.iota(jnp.int32, L)
    zero_v = jnp.zeros((L,), jnp.float32)
    sems = (sem0, sem1)

    def splat_last(s):
        return jnp.take_along_axis(s, last, axis=0, mode="promise_in_bounds")

    def splat_lane(s, j):
        return jnp.take_along_axis(s, lane_c[j], axis=0,
                                   mode="promise_in_bounds")

    def do_group(grp, _):
        row0 = base_row + grp * G

        def issue_inputs(slot, c):
            for g in range(G):
                pltpu.async_copy(
                    x_hbm.at[row0 + g, pl.ds(c * CHUNK, CHUNK)],
                    xbuf.at[slot, g], sems[slot])
                pltpu.async_copy(
                    m_hbm.at[row0 + g, pl.ds(c * (CHUNK // 4), CHUNK // 4)],
                    mbuf.at[slot, g], sems[slot])

        def wait_inputs(slot):
            for g in range(G):
                pltpu.make_async_copy(
                    x_hbm.at[row0 + g, pl.ds(0, CHUNK)],
                    xbuf.at[slot, g], sems[slot]).wait()
                pltpu.make_async_copy(
                    m_hbm.at[row0 + g, pl.ds(0, CHUNK // 4)],
                    mbuf.at[slot, g], sems[slot]).wait()

        def drain_outputs(slot):
            for g in range(G):
                pltpu.make_async_copy(
                    obuf.at[slot, g],
                    out_hbm.at[row0 + g, pl.ds(0, CHUNK)], sem_out).wait()

        issue_inputs(0, 0)

        def do_pair(cc, carries):
            for par in range(2):
                c = cc * 2 + par
                slot, other = par, 1 - par

                @pl.when(c < NCH - 1)
                def _():
                    issue_inputs(other, c + 1)
                wait_inputs(slot)
                @pl.when(c >= 2)
                def _():
                    drain_outputs(slot)

                slot_c = jnp.full((L,), slot, jnp.int32)

                def do_blk(b, cs):
                    cs = list(cs)
                    col0 = b * BLK
                    w0 = b * (BLK // 4)
                    for g in range(G):
                        ws = [mbuf[slot, g, pl.ds(w0 + q * L, L)]
                              for q in range(4)]
                        for j in range(L):
                            sl = pl.ds(col0 + j * L, L)
                            q, k = j // 4, j % 4
                            w = ws[q]
                            mbits = (w >> (8 * k)) & 0xFF if k else w & 0xFF
                            v = jnp.where(mbits != 0, xbuf[slot, g, sl],
                                          zero_v)
                            xbuf[slot, g, sl] = plsc.cumsum(v)
                        g_c = jnp.full((L,), g, jnp.int32)
                        idx = iota16 * L + (col0 + L - 1)
                        t = plsc.load_gather(xbuf, [slot_c, g_c, idx])
                        T = plsc.cumsum(t)
                        excl = T - t + cs[g]
                        cs[g] = splat_last(T) + cs[g]
                        for j in range(L):
                            sl = pl.ds(col0 + j * L, L)
                            obuf[slot, g, sl] = (xbuf[slot, g, sl]
                                                 + splat_lane(excl, j))
                    return tuple(cs)

                carries = lax.fori_loop(0, NB, do_blk, carries)

                c0 = c * CHUNK
                for g in range(G):
                    pltpu.async_copy(
                        obuf.at[slot, g],
                        out_hbm.at[row0 + g, pl.ds(c0, CHUNK)], sem_out)
            return carries

        zeros = tuple(zero_v for _ in range(G))
        lax.fori_loop(0, NCH // 2, do_pair, zeros)

        drain_outputs(0)
        drain_outputs(1)
        return 0

    lax.fori_loop(0, NGRP, do_group, 0)


def kernel(x, mask):
    m8 = mask.astype(jnp.int8)
    m8 = m8.reshape(ROWS, COLS // 64, 4, 16).transpose(0, 1, 3, 2)
    m32 = jax.lax.bitcast_convert_type(m8.reshape(ROWS, COLS // 4, 4),
                                       jnp.int32)
    return _masked_cumsum_sc(x, m32)


# speedup vs baseline: 1.7324x; 1.7324x over previous
import functools

import jax
import jax.numpy as jnp
from jax import lax
from jax.experimental import pallas as pl
from jax.experimental.pallas import tpu as pltpu
from jax.experimental.pallas import tpu_sc as plsc

ROWS, COLS = 1024, 32768
NC, NS, L = 2, 16, 16
NW = NC * NS
ROWS_PER_W = ROWS // NW
G = 8
NGRP = ROWS_PER_W // G
CHUNK = 2048
NCH = COLS // CHUNK

_MESH = plsc.VectorSubcoreMesh(
    core_axis_name="c", subcore_axis_name="s", num_cores=NC, num_subcores=NS
)


@functools.partial(
    pl.kernel,
    out_type=jax.ShapeDtypeStruct((ROWS, COLS), jnp.float32),
    mesh=_MESH,
    scratch_types=[
        pltpu.VMEM((2, G, CHUNK), jnp.float32),
        pltpu.VMEM((2, G, CHUNK), jnp.float32),
        pltpu.VMEM((2, G, CHUNK), jnp.float32),
        pltpu.SemaphoreType.DMA,
        pltpu.SemaphoreType.DMA,
        pltpu.SemaphoreType.DMA,
    ],
    compiler_params=pltpu.CompilerParams(needs_layout_passes=False),
)
def _masked_cumsum_sc(x_hbm, m_hbm, out_hbm, xbuf, mbuf, obuf,
                      sem0, sem1, sem_out):
    wid = lax.axis_index("s") * NC + lax.axis_index("c")
    base_row = wid * ROWS_PER_W
    last = jnp.full((L,), L - 1, jnp.int32)
    sems = (sem0, sem1)

    def splat_last(s):
        return jnp.take_along_axis(s, last, axis=0, mode="promise_in_bounds")

    def do_group(grp, _):
        row0 = base_row + grp * G

        def issue_inputs(slot, c):
            c0 = c * CHUNK
            for g in range(G):
                pltpu.async_copy(
                    x_hbm.at[row0 + g, pl.ds(c0, CHUNK)],
                    xbuf.at[slot, g], sems[slot])
                pltpu.async_copy(
                    m_hbm.at[row0 + g, pl.ds(c0, CHUNK)],
                    mbuf.at[slot, g], sems[slot])

        def wait_inputs(slot):
            for g in range(G):
                pltpu.make_async_copy(
                    x_hbm.at[row0 + g, pl.ds(0, CHUNK)],
                    xbuf.at[slot, g], sems[slot]).wait()
                pltpu.make_async_copy(
                    m_hbm.at[row0 + g, pl.ds(0, CHUNK)],
                    mbuf.at[slot, g], sems[slot]).wait()

        def drain_outputs(slot):
            for g in range(G):
                pltpu.make_async_copy(
                    obuf.at[slot, g],
                    out_hbm.at[row0 + g, pl.ds(0, CHUNK)], sem_out).wait()

        issue_inputs(0, 0)

        def do_pair(cc, carries):
            for par in range(2):
                c = cc * 2 + par
                slot, other = par, 1 - par

                @pl.when(c < NCH - 1)
                def _():
                    issue_inputs(other, c + 1)
                wait_inputs(slot)
                @pl.when(c >= 2)
                def _():
                    drain_outputs(slot)

                def do_vec(j, cs):
                    sl = pl.ds(j * L, L)
                    out = []
                    for g in range(G):
                        v = xbuf[slot, g, sl] * mbuf[slot, g, sl]
                        s = plsc.cumsum(v) + cs[g]
                        obuf[slot, g, sl] = s
                        out.append(splat_last(s))
                    return tuple(out)

                carries = lax.fori_loop(0, CHUNK // L, do_vec, carries,
                                        unroll=2)

                c0 = c * CHUNK
                for g in range(G):
                    pltpu.async_copy(
                        obuf.at[slot, g],
                        out_hbm.at[row0 + g, pl.ds(c0, CHUNK)], sem_out)
            return carries

        zeros = tuple(jnp.zeros((L,), jnp.float32) for _ in range(G))
        lax.fori_loop(0, NCH // 2, do_pair, zeros)

        drain_outputs(0)
        drain_outputs(1)
        return 0

    lax.fori_loop(0, NGRP, do_group, 0)


def kernel(x, mask):
    return _masked_cumsum_sc(x, mask.astype(jnp.float32))


# speedup vs baseline: 1.9736x; 1.1392x over previous
import functools

import jax
import jax.numpy as jnp
from jax import lax
from jax.experimental import pallas as pl
from jax.experimental.pallas import tpu as pltpu
from jax.experimental.pallas import tpu_sc as plsc

ROWS, COLS = 1024, 32768
SC_ROWS = 512
TC_ROWS = ROWS - SC_ROWS

NC, NS, L = 2, 16, 16
NW = NC * NS
ROWS_PER_W = SC_ROWS // NW
G = 8
NGRP = ROWS_PER_W // G
CHUNK = 2048
NCH = COLS // CHUNK

_MESH = plsc.VectorSubcoreMesh(
    core_axis_name="c", subcore_axis_name="s", num_cores=NC, num_subcores=NS
)


@functools.partial(
    pl.kernel,
    out_type=jax.ShapeDtypeStruct((SC_ROWS, COLS), jnp.float32),
    mesh=_MESH,
    scratch_types=[
        pltpu.VMEM((2, G, CHUNK), jnp.float32),
        pltpu.VMEM((2, G, CHUNK), jnp.float32),
        pltpu.VMEM((2, G, CHUNK), jnp.float32),
        pltpu.SemaphoreType.DMA,
        pltpu.SemaphoreType.DMA,
        pltpu.SemaphoreType.DMA,
    ],
    compiler_params=pltpu.CompilerParams(needs_layout_passes=False),
)
def _masked_cumsum_sc(x_hbm, m_hbm, out_hbm, xbuf, mbuf, obuf,
                      sem0, sem1, sem_out):
    wid = lax.axis_index("s") * NC + lax.axis_index("c")
    base_row = wid * ROWS_PER_W
    last = jnp.full((L,), L - 1, jnp.int32)
    sems = (sem0, sem1)

    def splat_last(s):
        return jnp.take_along_axis(s, last, axis=0, mode="promise_in_bounds")

    def do_group(grp, _):
        row0 = base_row + grp * G

        def issue_inputs(slot, c):
            c0 = c * CHUNK
            for g in range(G):
                pltpu.async_copy(
                    x_hbm.at[row0 + g, pl.ds(c0, CHUNK)],
                    xbuf.at[slot, g], sems[slot])
                pltpu.async_copy(
                    m_hbm.at[row0 + g, pl.ds(c0, CHUNK)],
                    mbuf.at[slot, g], sems[slot])

        def wait_inputs(slot):
            for g in range(G):
                pltpu.make_async_copy(
                    x_hbm.at[row0 + g, pl.ds(0, CHUNK)],
                    xbuf.at[slot, g], sems[slot]).wait()
                pltpu.make_async_copy(
                    m_hbm.at[row0 + g, pl.ds(0, CHUNK)],
                    mbuf.at[slot, g], sems[slot]).wait()

        def drain_outputs(slot):
            for g in range(G):
                pltpu.make_async_copy(
                    obuf.at[slot, g],
                    out_hbm.at[row0 + g, pl.ds(0, CHUNK)], sem_out).wait()

        issue_inputs(0, 0)

        def do_pair(cc, carries):
            for par in range(2):
                c = cc * 2 + par
                slot, other = par, 1 - par

                @pl.when(c < NCH - 1)
                def _():
                    issue_inputs(other, c + 1)
                wait_inputs(slot)
                @pl.when(c >= 2)
                def _():
                    drain_outputs(slot)

                def do_vec(j, cs):
                    sl = pl.ds(j * L, L)
                    out = []
                    for g in range(G):
                        v = xbuf[slot, g, sl] * mbuf[slot, g, sl]
                        s = plsc.cumsum(v) + cs[g]
                        obuf[slot, g, sl] = s
                        out.append(splat_last(s))
                    return tuple(out)

                carries = lax.fori_loop(0, CHUNK // L, do_vec, carries,
                                        unroll=2)

                c0 = c * CHUNK
                for g in range(G):
                    pltpu.async_copy(
                        obuf.at[slot, g],
                        out_hbm.at[row0 + g, pl.ds(c0, CHUNK)], sem_out)
            return carries

        zeros = tuple(jnp.zeros((L,), jnp.float32) for _ in range(G))
        lax.fori_loop(0, NCH // 2, do_pair, zeros)

        drain_outputs(0)
        drain_outputs(1)
        return 0

    lax.fori_loop(0, NGRP, do_group, 0)


BR = 128
CB = 4096
NCHK = CB // 128


def _tc_body(x_ref, m_ref, out_ref, carry_ref):
    col = pl.program_id(1)

    @pl.when(col == 0)
    def _():
        carry_ref[...] = jnp.zeros_like(carry_ref)

    z = jnp.where(m_ref[...], x_ref[...], 0.0)
    z3 = z.reshape(BR, NCHK, 128)
    tri = (lax.broadcasted_iota(jnp.int32, (128, 128), 0)
           <= lax.broadcasted_iota(jnp.int32, (128, 128), 1))
    local = jax.lax.dot_general(
        z3, tri.astype(jnp.float32),
        dimension_numbers=(((2,), (0,)), ((), ())),
        preferred_element_type=jnp.float32,
        precision=lax.Precision.HIGHEST)
    totals = z3.sum(axis=2)
    stri = (lax.broadcasted_iota(jnp.int32, (NCHK, NCHK), 0)
            < lax.broadcasted_iota(jnp.int32, (NCHK, NCHK), 1))
    offs = jax.lax.dot_general(
        totals, stri.astype(jnp.float32),
        dimension_numbers=(((1,), (0,)), ((), ())),
        preferred_element_type=jnp.float32,
        precision=lax.Precision.HIGHEST)
    offs = offs + carry_ref[...]
    out_ref[...] = (local + offs[:, :, None]).reshape(BR, CB)
    carry_ref[...] = carry_ref[...] + totals.sum(axis=1, keepdims=True)


_masked_cumsum_tc = pl.pallas_call(
    _tc_body,
    out_shape=jax.ShapeDtypeStruct((TC_ROWS, COLS), jnp.float32),
    grid=(TC_ROWS // BR, COLS // CB),
    in_specs=[
        pl.BlockSpec((BR, CB), lambda r, c: (r, c)),
        pl.BlockSpec((BR, CB), lambda r, c: (r, c)),
    ],
    out_specs=pl.BlockSpec((BR, CB), lambda r, c: (r, c)),
    scratch_shapes=[pltpu.VMEM((BR, 1), jnp.float32)],
    compiler_params=pltpu.CompilerParams(
        dimension_semantics=("parallel", "arbitrary")),
)


def kernel(x, mask):
    sc_out = _masked_cumsum_sc(x[:SC_ROWS],
                               mask[:SC_ROWS].astype(jnp.float32))
    tc_out = _masked_cumsum_tc(x[SC_ROWS:], mask[SC_ROWS:])
    return jnp.concatenate([sc_out, tc_out], axis=0)
